# baseline (speedup 1.0000x reference)
import functools

import jax
import jax.numpy as jnp
from jax import lax
from jax.experimental import pallas as pl
from jax.experimental.pallas import tpu as pltpu
from jax.experimental.pallas import tpu_sc as plsc

_NUM_CORES = 2
_NUM_SUBCORES = 16
_LANES = 16
_NW = _NUM_CORES * _NUM_SUBCORES

_ROWS = 200
_COLS = 16384
_STRIPE = 40
_NCHUNK = _ROWS // _STRIPE
_CCOL = _COLS // _NW
_TPAD = 128
_DDIM = 10


def _sc_body(e1t_hbm, e2t_hbm, w_hbm, b_hbm, x_hbm, out_hbm,
             e1t_v, e2t_v, w_v, b_v, table_v,
             idx_v, res_v, in_sems, out_sems, stage_sem):
  wid = lax.axis_index("s") * _NUM_CORES + lax.axis_index("c")
  col0 = wid * _CCOL

  def start_in(c, buf):
    return pltpu.async_copy(
        x_hbm.at[pl.ds(c * _STRIPE, _STRIPE), pl.ds(col0, _CCOL)],
        idx_v.at[buf], in_sems.at[buf])

  in_copies = [start_in(0, 0), start_in(1, 1)]

  stage = [pltpu.async_copy(e1t_hbm, e1t_v, stage_sem),
           pltpu.async_copy(e2t_hbm, e2t_v, stage_sem),
           pltpu.async_copy(w_hbm, w_v, stage_sem),
           pltpu.async_copy(b_hbm, b_v, stage_sem)]
  for cp in stage:
    cp.wait()

  for jc in range(_TPAD // _LANES):
    sl = pl.ds(jc * _LANES, _LANES)
    acc = b_v[:]
    for d in range(_DDIM):
      acc = acc + (e1t_v[d, sl] + e2t_v[d, sl]) * w_v[d, :]
    table_v[sl] = acc

  def start_out(c, buf):
    return pltpu.async_copy(
        res_v.at[buf],
        out_hbm.at[pl.ds(c * _STRIPE, _STRIPE), pl.ds(col0, _CCOL)],
        out_sems.at[buf])

  def compute(buf):
    @plsc.parallel_loop(0, _STRIPE, step=1, unroll=2)
    def _(u):
      for k in range(_CCOL // _LANES):
        sl = pl.ds(k * _LANES, _LANES)
        res_v[buf, u, sl] = plsc.load_gather(table_v, [idx_v[buf, u, sl]])

  out_copies = [None, None]
  for c in range(_NCHUNK):
    buf = c % 2
    in_copies[buf].wait()
    if out_copies[buf] is not None:
      out_copies[buf].wait()
    compute(buf)
    out_copies[buf] = start_out(c, buf)
    if c + 2 < _NCHUNK:
      in_copies[buf] = start_in(c + 2, buf)
  out_copies[(_NCHUNK - 2) % 2].wait()
  out_copies[(_NCHUNK - 1) % 2].wait()


@jax.jit
def _run(xt, e1t, e2t, w_rep, b_rep):
  mesh = plsc.VectorSubcoreMesh(
      core_axis_name="c", subcore_axis_name="s",
      num_cores=_NUM_CORES, num_subcores=_NUM_SUBCORES)
  kern = functools.partial(
      pl.kernel,
      out_type=jax.ShapeDtypeStruct((_ROWS, _COLS), jnp.float32),
      mesh=mesh,
      scratch_types=[
          pltpu.VMEM((_DDIM, _TPAD), jnp.float32),
          pltpu.VMEM((_DDIM, _TPAD), jnp.float32),
          pltpu.VMEM((_DDIM, _LANES), jnp.float32),
          pltpu.VMEM((_LANES,), jnp.float32),
          pltpu.VMEM((_TPAD,), jnp.float32),
          pltpu.VMEM((2, _STRIPE, _CCOL), jnp.int32),
          pltpu.VMEM((2, _STRIPE, _CCOL), jnp.float32),
          pltpu.SemaphoreType.DMA((2,)),
          pltpu.SemaphoreType.DMA((2,)),
          pltpu.SemaphoreType.DMA,
      ],
      compiler_params=pltpu.CompilerParams(needs_layout_passes=False),
  )(_sc_body)
  return kern(e1t, e2t, w_rep, b_rep, xt)


def kernel(x, embed_in, embed_in_2, lin0_w, lin0_b):
  xt = x.T.astype(jnp.int32)
  e1t = jnp.pad(embed_in.T, ((0, 0), (0, _TPAD - embed_in.shape[0])))
  e2t = jnp.pad(embed_in_2.T, ((0, 0), (0, _TPAD - embed_in_2.shape[0])))
  w_rep = jnp.broadcast_to(lin0_w.reshape(_DDIM, 1), (_DDIM, _LANES))
  b_rep = jnp.broadcast_to(lin0_b.reshape(1), (_LANES,))
  out_t = _run(xt, e1t, e2t, w_rep, b_rep)
  return out_t.T[:, :, None]


# speedup vs baseline: 1.1178x; 1.1178x over previous
import functools

import jax
import jax.numpy as jnp
from jax import lax
from jax.experimental import pallas as pl
from jax.experimental.pallas import tpu as pltpu
from jax.experimental.pallas import tpu_sc as plsc

_NUM_CORES = 2
_NUM_SUBCORES = 16
_LANES = 16
_NW = _NUM_CORES * _NUM_SUBCORES

_ROWS = 200
_COLS = 16384
_STRIPE = 40
_NCHUNK = _ROWS // _STRIPE
_CCOL = _COLS // _NW
_TPAD = 128
_DDIM = 10


def _sc_body(e1t_hbm, e2t_hbm, w_hbm, b_hbm, x_hbm, out_hbm,
             e1t_v, e2t_v, w_v, b_v, table_v,
             idx_v, res_v, in_sems, out_sems, stage_sem):
  wid = lax.axis_index("s") * _NUM_CORES + lax.axis_index("c")
  col0 = wid * _CCOL

  def start_in(c, buf):
    return pltpu.async_copy(
        x_hbm.at[pl.ds(c * _STRIPE, _STRIPE), pl.ds(col0, _CCOL)],
        idx_v.at[buf], in_sems.at[buf])

  in_copies = [start_in(0, 0), start_in(1, 1)]

  stage = [pltpu.async_copy(e1t_hbm, e1t_v, stage_sem),
           pltpu.async_copy(e2t_hbm, e2t_v, stage_sem),
           pltpu.async_copy(w_hbm, w_v, stage_sem),
           pltpu.async_copy(b_hbm, b_v, stage_sem)]
  for cp in stage:
    cp.wait()

  for jc in range(_TPAD // _LANES):
    sl = pl.ds(jc * _LANES, _LANES)
    acc = b_v[:]
    for d in range(_DDIM):
      acc = acc + (e1t_v[d, sl] + e2t_v[d, sl]) * w_v[d, :]
    table_v[sl] = acc

  def start_out(c, buf):
    return pltpu.async_copy(
        res_v.at[buf],
        out_hbm.at[pl.ds(c * _STRIPE, _STRIPE), pl.ds(col0, _CCOL)],
        out_sems.at[buf])

  def compute(buf):
    @plsc.parallel_loop(0, _STRIPE * _CCOL, step=_LANES, unroll=16)
    def _(i):
      u = i // _CCOL
      sl = pl.ds(pl.multiple_of(i % _CCOL, _LANES), _LANES)
      res_v[buf, u, sl] = plsc.load_gather(table_v, [idx_v[buf, u, sl]])

  out_copies = [None, None]
  for c in range(_NCHUNK):
    buf = c % 2
    in_copies[buf].wait()
    if out_copies[buf] is not None:
      out_copies[buf].wait()
    compute(buf)
    out_copies[buf] = start_out(c, buf)
    if c + 2 < _NCHUNK:
      in_copies[buf] = start_in(c + 2, buf)
  out_copies[(_NCHUNK - 2) % 2].wait()
  out_copies[(_NCHUNK - 1) % 2].wait()


@jax.jit
def _run(xt, e1t, e2t, w_rep, b_rep):
  mesh = plsc.VectorSubcoreMesh(
      core_axis_name="c", subcore_axis_name="s",
      num_cores=_NUM_CORES, num_subcores=_NUM_SUBCORES)
  kern = functools.partial(
      pl.kernel,
      out_type=jax.ShapeDtypeStruct((_ROWS, _COLS), jnp.float32),
      mesh=mesh,
      scratch_types=[
          pltpu.VMEM((_DDIM, _TPAD), jnp.float32),
          pltpu.VMEM((_DDIM, _TPAD), jnp.float32),
          pltpu.VMEM((_DDIM, _LANES), jnp.float32),
          pltpu.VMEM((_LANES,), jnp.float32),
          pltpu.VMEM((_TPAD,), jnp.float32),
          pltpu.VMEM((2, _STRIPE, _CCOL), jnp.int32),
          pltpu.VMEM((2, _STRIPE, _CCOL), jnp.float32),
          pltpu.SemaphoreType.DMA((2,)),
          pltpu.SemaphoreType.DMA((2,)),
          pltpu.SemaphoreType.DMA,
      ],
      compiler_params=pltpu.CompilerParams(needs_layout_passes=False),
  )(_sc_body)
  return kern(e1t, e2t, w_rep, b_rep, xt)


def kernel(x, embed_in, embed_in_2, lin0_w, lin0_b):
  xt = x.T.astype(jnp.int32)
  e1t = jnp.pad(embed_in.T, ((0, 0), (0, _TPAD - embed_in.shape[0])))
  e2t = jnp.pad(embed_in_2.T, ((0, 0), (0, _TPAD - embed_in_2.shape[0])))
  w_rep = jnp.broadcast_to(lin0_w.reshape(_DDIM, 1), (_DDIM, _LANES))
  b_rep = jnp.broadcast_to(lin0_b.reshape(1), (_LANES,))
  out_t = _run(xt, e1t, e2t, w_rep, b_rep)
  return out_t.T[:, :, None]


# speedup vs baseline: 1.1271x; 1.0083x over previous
import functools

import jax
import jax.numpy as jnp
from jax import lax
from jax.experimental import pallas as pl
from jax.experimental.pallas import tpu as pltpu
from jax.experimental.pallas import tpu_sc as plsc

_NUM_CORES = 2
_NUM_SUBCORES = 16
_LANES = 16
_NW = _NUM_CORES * _NUM_SUBCORES

_ROWS = 200
_COLS = 16384
_STRIPE = 40
_NCHUNK = _ROWS // _STRIPE
_CCOL = _COLS // _NW
_TPAD = 128
_DDIM = 10


def _sc_body(e1t_hbm, e2t_hbm, w_hbm, b_hbm, x_hbm, out_hbm,
             e1t_v, e2t_v, w_v, b_v, table_v,
             idx_v, res_v, in_sems, out_sems, stage_sem):
  wid = lax.axis_index("s") * _NUM_CORES + lax.axis_index("c")
  col0 = wid * _CCOL

  def start_in(c, buf):
    return pltpu.async_copy(
        x_hbm.at[pl.ds(c * _STRIPE, _STRIPE), pl.ds(col0, _CCOL)],
        idx_v.at[buf], in_sems.at[buf])

  in_copies = [start_in(0, 0), start_in(1, 1)]

  stage = [pltpu.async_copy(e1t_hbm, e1t_v, stage_sem),
           pltpu.async_copy(e2t_hbm, e2t_v, stage_sem),
           pltpu.async_copy(w_hbm, w_v, stage_sem),
           pltpu.async_copy(b_hbm, b_v, stage_sem)]
  for cp in stage:
    cp.wait()

  for jc in range(_TPAD // _LANES):
    sl = pl.ds(jc * _LANES, _LANES)
    acc = b_v[:]
    for d in range(_DDIM):
      acc = acc + (e1t_v[d, sl] + e2t_v[d, sl]) * w_v[d, :]
    table_v[sl] = acc

  def start_out(c, buf):
    return pltpu.async_copy(
        res_v.at[buf],
        out_hbm.at[pl.ds(c * _STRIPE, _STRIPE), pl.ds(col0, _CCOL)],
        out_sems.at[buf])

  def compute(buf):
    @plsc.parallel_loop(0, _STRIPE * _CCOL, step=_LANES, unroll=8)
    def _(i):
      u = i // _CCOL
      sl = pl.ds(pl.multiple_of(i % _CCOL, _LANES), _LANES)
      res_v[buf, u, sl] = plsc.load_gather(table_v, [idx_v[buf, u, sl]])

  out_copies = [None, None]
  for c in range(_NCHUNK):
    buf = c % 2
    in_copies[buf].wait()
    if out_copies[buf] is not None:
      out_copies[buf].wait()
    compute(buf)
    out_copies[buf] = start_out(c, buf)
    if c + 2 < _NCHUNK:
      in_copies[buf] = start_in(c + 2, buf)
  out_copies[(_NCHUNK - 2) % 2].wait()
  out_copies[(_NCHUNK - 1) % 2].wait()


@jax.jit
def _run(xt, e1t, e2t, w_rep, b_rep):
  mesh = plsc.VectorSubcoreMesh(
      core_axis_name="c", subcore_axis_name="s",
      num_cores=_NUM_CORES, num_subcores=_NUM_SUBCORES)
  kern = functools.partial(
      pl.kernel,
      out_type=jax.ShapeDtypeStruct((_ROWS, _COLS), jnp.float32),
      mesh=mesh,
      scratch_types=[
          pltpu.VMEM((_DDIM, _TPAD), jnp.float32),
          pltpu.VMEM((_DDIM, _TPAD), jnp.float32),
          pltpu.VMEM((_DDIM, _LANES), jnp.float32),
          pltpu.VMEM((_LANES,), jnp.float32),
          pltpu.VMEM((_TPAD,), jnp.float32),
          pltpu.VMEM((2, _STRIPE, _CCOL), jnp.int32),
          pltpu.VMEM((2, _STRIPE, _CCOL), jnp.float32),
          pltpu.SemaphoreType.DMA((2,)),
          pltpu.SemaphoreType.DMA((2,)),
          pltpu.SemaphoreType.DMA,
      ],
      compiler_params=pltpu.CompilerParams(needs_layout_passes=False),
  )(_sc_body)
  return kern(e1t, e2t, w_rep, b_rep, xt)


def kernel(x, embed_in, embed_in_2, lin0_w, lin0_b):
  xt = x.T.astype(jnp.int32)
  e1t = jnp.pad(embed_in.T, ((0, 0), (0, _TPAD - embed_in.shape[0])))
  e2t = jnp.pad(embed_in_2.T, ((0, 0), (0, _TPAD - embed_in_2.shape[0])))
  w_rep = jnp.broadcast_to(lin0_w.reshape(_DDIM, 1), (_DDIM, _LANES))
  b_rep = jnp.broadcast_to(lin0_b.reshape(1), (_LANES,))
  out_t = _run(xt, e1t, e2t, w_rep, b_rep)
  return out_t.T[:, :, None]
